# Initial kernel scaffold; baseline (speedup 1.0000x reference)
import jax
import jax.numpy as jnp
from jax.experimental import pallas as pl


def kernel(distance_matrix, distance_embedding):
    raise NotImplementedError("write your pallas kernel here")


# speedup vs baseline: 29.4857x; 29.4857x over previous
import functools

import jax
import jax.numpy as jnp
from jax import lax
from jax.experimental import pallas as pl
from jax.experimental.pallas import tpu as pltpu
from jax.experimental.pallas import tpu_sc as plsc

_B = 256
_N = 128
_H = 8
_V = 129
_PLANE = _N * _N
_CHUNK = 4096
_VECS = _CHUNK // 16
_NCHUNK = _PLANE // _CHUNK


@functools.cache
def _build_sc_kernel():
    info = plsc.get_sparse_core_info()
    nc, ns = info.num_cores, info.num_subcores
    nw = nc * ns
    bpw = _B // nw
    mesh = plsc.VectorSubcoreMesh(core_axis_name="c", subcore_axis_name="s")

    @functools.partial(
        pl.kernel,
        mesh=mesh,
        out_type=jax.ShapeDtypeStruct((_B, _H, _PLANE), jnp.float32),
        compiler_params=pltpu.CompilerParams(needs_layout_passes=False),
        scratch_types=[
            pltpu.VMEM((_H * _V,), jnp.float32),
            pltpu.VMEM((_CHUNK,), jnp.int32),
            pltpu.VMEM((_H, _CHUNK), jnp.float32),
        ],
    )
    def sc_kernel(dm_hbm, tab_hbm, out_hbm, tab_v, idx_v, out_v):
        wid = lax.axis_index("s") * nc + lax.axis_index("c")
        pltpu.sync_copy(tab_hbm, tab_v)
        for bi in range(bpw):
            b = wid * bpw + bi
            for ci in range(_NCHUNK):
                pltpu.sync_copy(dm_hbm.at[b, pl.ds(ci * _CHUNK, _CHUNK)], idx_v)

                def body(v, carry):
                    idx = idx_v[pl.ds(v * 16, 16)]
                    for h in range(_H):
                        val = plsc.load_gather(tab_v, [idx + h * _V])
                        out_v[h, pl.ds(v * 16, 16)] = val
                    return carry

                lax.fori_loop(0, _VECS, body, 0)
                for h in range(_H):
                    pltpu.sync_copy(
                        out_v.at[h],
                        out_hbm.at[b, h, pl.ds(ci * _CHUNK, _CHUNK)],
                    )

    return sc_kernel


def kernel(distance_matrix, distance_embedding):
    dm = distance_matrix
    if dm.dtype != jnp.int32:
        dm = dm.astype(jnp.int32)
    dm_flat = dm.reshape(_B, _PLANE)
    tab = distance_embedding.T.reshape(_H * _V)
    out = _build_sc_kernel()(dm_flat, tab)
    return out.reshape(_B, _H, _N, _N)


# speedup vs baseline: 64.1687x; 2.1763x over previous
import functools

import jax
import jax.numpy as jnp
from jax import lax
from jax.experimental import pallas as pl
from jax.experimental.pallas import tpu as pltpu
from jax.experimental.pallas import tpu_sc as plsc

_B = 256
_N = 128
_H = 8
_V = 129
_PLANE = _N * _N
_CHUNK = 4096
_VECS = _CHUNK // 16
_NCHUNK = _PLANE // _CHUNK


@functools.cache
def _build_sc_kernel():
    info = plsc.get_sparse_core_info()
    nc, ns = info.num_cores, info.num_subcores
    nw = nc * ns
    bpw = _B // nw
    mesh = plsc.VectorSubcoreMesh(core_axis_name="c", subcore_axis_name="s")

    @functools.partial(
        pl.kernel,
        mesh=mesh,
        out_type=jax.ShapeDtypeStruct((_B, _H, _PLANE), jnp.float32),
        compiler_params=pltpu.CompilerParams(needs_layout_passes=False),
        scratch_types=[
            pltpu.VMEM((_H * _V,), jnp.float32),
            pltpu.VMEM((2, _CHUNK), jnp.int32),
            pltpu.VMEM((2, _H, _CHUNK), jnp.float32),
            pltpu.SemaphoreType.DMA((2,)),
            pltpu.SemaphoreType.DMA((2,)),
        ],
    )
    def sc_kernel(dm_hbm, tab_hbm, out_hbm, tab_v, idx_v, out_v, in_sem, out_sem):
        wid = lax.axis_index("s") * nc + lax.axis_index("c")
        pltpu.sync_copy(tab_hbm, tab_v)
        steps = [(bi, ci) for bi in range(bpw) for ci in range(_NCHUNK)]
        nst = len(steps)

        def in_copy(t, buf):
            bi, ci = steps[t]
            b = wid * bpw + bi
            return pltpu.async_copy(
                dm_hbm.at[b, pl.ds(ci * _CHUNK, _CHUNK)],
                idx_v.at[buf],
                in_sem.at[buf],
            )

        def out_copies(t, buf):
            bi, ci = steps[t]
            b = wid * bpw + bi
            return [
                pltpu.async_copy(
                    out_v.at[buf, h],
                    out_hbm.at[b, h, pl.ds(ci * _CHUNK, _CHUNK)],
                    out_sem.at[buf],
                )
                for h in range(_H)
            ]

        pending = {}
        ic = in_copy(0, 0)
        for t in range(nst):
            cur = t & 1
            nxt_ic = in_copy(t + 1, 1 - cur) if t + 1 < nst else None
            ic.wait()
            if t >= 2:
                for c in pending.pop(t - 2):
                    c.wait()

            @plsc.parallel_loop(0, _VECS, unroll=2)
            def body(v):
                idx = idx_v[cur, pl.ds(v * 16, 16)]
                for h in range(_H):
                    val = plsc.load_gather(tab_v, [idx + h * _V])
                    out_v[cur, h, pl.ds(v * 16, 16)] = val

            pending[t] = out_copies(t, cur)
            ic = nxt_ic
        for t in (nst - 2, nst - 1):
            for c in pending.pop(t):
                c.wait()

    return sc_kernel


def kernel(distance_matrix, distance_embedding):
    dm = distance_matrix
    if dm.dtype != jnp.int32:
        dm = dm.astype(jnp.int32)
    dm_flat = dm.reshape(_B, _PLANE)
    tab = distance_embedding.T.reshape(_H * _V)
    out = _build_sc_kernel()(dm_flat, tab)
    return out.reshape(_B, _H, _N, _N)
